# baseline (speedup 1.0000x reference)
import functools

import jax
import jax.numpy as jnp
from jax.experimental import pallas as pl
from jax.experimental.pallas import tpu as pltpu

NUM_EXPERTS = 8
TOP_K = 2
N_EMBD = 1024
D_FFN = 2048


def _router_body(x_ref, wr_ref, wts_ref):
    x = x_ref[...]
    wr = wr_ref[...]
    logits = jax.lax.dot_general(
        x, wr, (((1,), (1,)), ((), ())), preferred_element_type=jnp.float32
    )
    m = jnp.max(logits, axis=-1, keepdims=True)
    ex = jnp.exp(logits - m)
    probs = ex / jnp.sum(ex, axis=-1, keepdims=True)
    e_iota = jax.lax.broadcasted_iota(jnp.int32, probs.shape, 1)
    m1 = jnp.max(probs, axis=-1, keepdims=True)
    i1 = jnp.min(jnp.where(probs == m1, e_iota, NUM_EXPERTS), axis=-1, keepdims=True)
    masked = jnp.where(e_iota == i1, -jnp.inf, probs)
    m2 = jnp.max(masked, axis=-1, keepdims=True)
    i2 = jnp.min(jnp.where(masked == m2, e_iota, NUM_EXPERTS), axis=-1, keepdims=True)
    s = m1 + m2
    w1n = m1 / s
    w2n = m2 / s
    wts_ref[...] = jnp.where(e_iota == i1, w1n, 0.0) + jnp.where(e_iota == i2, w2n, 0.0)


def _moe_body(x_ref, w1_ref, w2_ref, wts_ref, out_ref):
    e = pl.program_id(1)
    h = jax.lax.dot_general(
        x_ref[...], w1_ref[0], (((1,), (0,)), ((), ())),
        preferred_element_type=jnp.float32,
    )
    y = jax.lax.dot_general(
        h, w2_ref[0], (((1,), (0,)), ((), ())),
        preferred_element_type=jnp.float32,
    )
    e_iota = jax.lax.broadcasted_iota(jnp.int32, wts_ref.shape, 1)
    we = jnp.sum(jnp.where(e_iota == e, wts_ref[...], 0.0), axis=-1, keepdims=True)
    y = y * we

    @pl.when(e == 0)
    def _():
        out_ref[...] = y

    @pl.when(e != 0)
    def _():
        out_ref[...] = out_ref[...] + y


def kernel(x, w_router, w1, w2):
    b, s, d = x.shape
    t = b * s
    xf = x.reshape(t, d)

    wts = pl.pallas_call(
        _router_body,
        out_shape=jax.ShapeDtypeStruct((t, NUM_EXPERTS), jnp.float32),
    )(xf, w_router)

    w1r = w1.reshape(d, NUM_EXPERTS, D_FFN).transpose(1, 0, 2)
    w2r = w2.reshape(NUM_EXPERTS, D_FFN, d)

    tm = 512
    grid = (t // tm, NUM_EXPERTS)
    out = pl.pallas_call(
        _moe_body,
        grid=grid,
        in_specs=[
            pl.BlockSpec((tm, d), lambda i, e: (i, 0)),
            pl.BlockSpec((1, d, D_FFN), lambda i, e: (e, 0, 0)),
            pl.BlockSpec((1, D_FFN, d), lambda i, e: (e, 0, 0)),
            pl.BlockSpec((tm, NUM_EXPERTS), lambda i, e: (i, 0)),
        ],
        out_specs=pl.BlockSpec((tm, d), lambda i, e: (i, 0)),
        out_shape=jax.ShapeDtypeStruct((t, d), jnp.float32),
        compiler_params=pltpu.CompilerParams(
            dimension_semantics=("arbitrary", "arbitrary"),
        ),
    )(xf, w1r, w2r, wts)

    return out.reshape(b, s, d)


# speedup vs baseline: 1.0568x; 1.0568x over previous
import functools

import jax
import jax.numpy as jnp
from jax import lax
from jax.experimental import pallas as pl
from jax.experimental.pallas import tpu as pltpu
from jax.experimental.pallas import tpu_sc as plsc

NUM_EXPERTS = 8
N_EMBD = 1024
D_FFN = 2048
BLK = 128
NB = 40
NR = NB * BLK
L = 16


def _router_body(x_ref, wr_ref, sel_ref, wn_ref):
    x = x_ref[...]
    wr = wr_ref[...]
    logits = lax.dot_general(
        x, wr, (((1,), (1,)), ((), ())), preferred_element_type=jnp.float32
    )
    m = jnp.max(logits, axis=-1, keepdims=True)
    ex = jnp.exp(logits - m)
    probs = ex / jnp.sum(ex, axis=-1, keepdims=True)
    e_iota = lax.broadcasted_iota(jnp.int32, probs.shape, 1)
    m1 = jnp.max(probs, axis=-1, keepdims=True)
    i1 = jnp.min(jnp.where(probs == m1, e_iota, NUM_EXPERTS), axis=-1, keepdims=True)
    masked = jnp.where(e_iota == i1, -jnp.inf, probs)
    m2 = jnp.max(masked, axis=-1, keepdims=True)
    i2 = jnp.min(jnp.where(masked == m2, e_iota, NUM_EXPERTS), axis=-1, keepdims=True)
    s = m1 + m2
    sel_ref[...] = jnp.concatenate([i1, i2], axis=1)
    wn_ref[...] = jnp.concatenate([m1 / s, m2 / s], axis=1)


def _index_body(sel_hbm, wn_hbm, slots_hbm, wvec_hbm, meta_hbm,
                selv, wnv, slotsv, wvecv, metav):
    nslots = sel_hbm.shape[0]
    nv = nslots // L
    wid = lax.axis_index("s") * 2 + lax.axis_index("c")
    iota = lax.iota(jnp.int32, L)

    @pl.when(wid < NUM_EXPERTS + 1)
    def _():
        pltpu.sync_copy(sel_hbm, selv)

        def count_step(j, acc):
            v = selv[pl.ds(j * L, L)]
            return tuple(
                acc[f] + jnp.where(v == f, 1, 0) for f in range(NUM_EXPERTS)
            )

        zero = jnp.zeros((L,), jnp.int32)
        acc = lax.fori_loop(0, nv, count_step, (zero,) * NUM_EXPERTS)
        counts = [jnp.sum(a) for a in acc]
        bcs = [(c + BLK - 1) // BLK for c in counts]

        @pl.when(wid < NUM_EXPERTS)
        def _():
            pltpu.sync_copy(wn_hbm, wnv)
            e = wid
            base = jnp.int32(0)
            for f in range(NUM_EXPERTS):
                base = base + jnp.where(f < e, bcs[f], 0)
            base = base * BLK
            my_count = jnp.int32(0)
            my_bc = jnp.int32(0)
            for f in range(NUM_EXPERTS):
                my_count = my_count + jnp.where(f == e, counts[f], 0)
                my_bc = my_bc + jnp.where(f == e, bcs[f], 0)

            def fill_step(j, carry):
                off = base + j * L
                slotsv[pl.ds(off, L)] = nslots + ((off + iota) & (BLK - 1))
                wvecv[pl.ds(off, L)] = jnp.zeros((L,), jnp.float32)
                return carry

            lax.fori_loop(0, my_bc * (BLK // L), fill_step, 0)

            def rank_step(j, cnt):
                sl = j * L + iota
                v = selv[pl.ds(j * L, L)]
                mk = v == e
                c = plsc.cumsum(jnp.where(mk, 1, 0))
                p = base + cnt + c - 1
                plsc.store_scatter(slotsv, [p], sl, mask=mk)
                wvals = wnv[pl.ds(j * L, L)]
                plsc.store_scatter(wvecv, [p], wvals, mask=mk)
                return cnt + jnp.where(mk, 1, 0).sum()

            lax.fori_loop(0, nv, rank_step, jnp.int32(0))

            def out_step(j, carry):
                off = base + j * BLK
                pltpu.sync_copy(slotsv.at[pl.ds(off, BLK)],
                                slots_hbm.at[pl.ds(off, BLK)])
                pltpu.sync_copy(wvecv.at[pl.ds(off, BLK)],
                                wvec_hbm.at[pl.ds(off, BLK)])
                return carry

            lax.fori_loop(0, my_bc, out_step, 0)

        @pl.when(wid == NUM_EXPERTS)
        def _():
            prefix = []
            run = jnp.int32(0)
            for f in range(NUM_EXPERTS):
                run = run + bcs[f]
                prefix.append(run)
            nact = prefix[-1]
            for v in range(4):
                blk = v * L + iota
                eid = jnp.zeros((L,), jnp.int32)
                for f in range(NUM_EXPERTS - 1):
                    eid = eid + jnp.where(blk >= prefix[f], 1, 0)
                eid = jnp.where(blk < nact, eid, 0)
                if v == 3:
                    eid = jnp.where(iota == 0, nact, 0)
                metav[pl.ds(v * L, L)] = eid
            pltpu.sync_copy(metav, meta_hbm)


def _gather_body(xf_hbm, slots_hbm, meta_hbm, xs_hbm,
                 sblk, tik, rowbuf, metav, sem):
    t_tokens = xf_hbm.shape[0]
    wid = lax.axis_index("s") * 2 + lax.axis_index("c")
    pltpu.sync_copy(meta_hbm.at[pl.ds(48, L)], metav)
    nact = jnp.max(metav[...])

    for rep in range(2):
        b = wid + rep * 32

        @pl.when(b < nact)
        def _():
            pltpu.sync_copy(slots_hbm.at[pl.ds(b * BLK, BLK)], sblk)
            for c in range(4):
                for h in range(2):
                    sv = sblk[pl.ds(c * 32 + h * L, L)]
                    tv = jnp.minimum(lax.shift_right_logical(sv, 1),
                                     t_tokens - 1)
                    tik[pl.ds(h * L, L)] = tv
                pltpu.async_copy(xf_hbm.at[tik], rowbuf, sem).wait()
                pltpu.sync_copy(
                    rowbuf, xs_hbm.at[pl.ds(b * BLK + c * 32, 32)])


def _mm_body(meta_ref, xs_ref, w1_ref, w2_ref, wv_ref, out_ref):
    b = pl.program_id(0)
    nact = meta_ref[48]

    @pl.when(b < nact)
    def _():
        xb = xs_ref[0].astype(jnp.bfloat16)
        h = lax.dot_general(
            xb, w1_ref[...], (((1,), (0,)), ((), ())),
            preferred_element_type=jnp.float32,
        ).astype(jnp.bfloat16)
        y = lax.dot_general(
            h, w2_ref[...], (((1,), (0,)), ((), ())),
            preferred_element_type=jnp.float32,
        )
        out_ref[0] = y * wv_ref[0]


def _scatter_body(ys_hbm, slots_hbm, meta_hbm, buf_hbm,
                  sblk, sidx, rowbuf, metav, sem):
    wid = lax.axis_index("s") * 2 + lax.axis_index("c")
    pltpu.sync_copy(meta_hbm.at[pl.ds(48, L)], metav)
    nact = jnp.max(metav[...])

    for rep in range(2):
        b = wid + rep * 32

        @pl.when(b < nact)
        def _():
            pltpu.sync_copy(slots_hbm.at[pl.ds(b * BLK, BLK)], sblk)
            for c in range(4):
                for h in range(2):
                    sidx[c, pl.ds(h * L, L)] = sblk[pl.ds(c * 32 + h * L, L)]
            for c in range(4):
                pltpu.sync_copy(ys_hbm.at[pl.ds(b * BLK + c * 32, 32)], rowbuf)
                pltpu.async_copy(rowbuf, buf_hbm.at[sidx.at[c]], sem).wait()


def _pair_body(buf_ref, out_ref):
    out_ref[...] = buf_ref[:, 0, :] + buf_ref[:, 1, :]


def kernel(x, w_router, w1, w2):
    b, s, d = x.shape
    t = b * s
    nslots = 2 * t
    xf = x.reshape(t, d)

    sel, wn = pl.pallas_call(
        _router_body,
        out_shape=(
            jax.ShapeDtypeStruct((t, 2), jnp.int32),
            jax.ShapeDtypeStruct((t, 2), jnp.float32),
        ),
    )(xf, w_router)

    sel_flat = sel.reshape(nslots)
    wn_flat = wn.reshape(nslots)

    mesh = plsc.VectorSubcoreMesh(core_axis_name="c", subcore_axis_name="s")
    sc_params = pltpu.CompilerParams(needs_layout_passes=False)

    slots, wvec, meta = pl.kernel(
        _index_body,
        out_type=(
            jax.ShapeDtypeStruct((NR,), jnp.int32),
            jax.ShapeDtypeStruct((NR,), jnp.float32),
            jax.ShapeDtypeStruct((64,), jnp.int32),
        ),
        mesh=mesh,
        scratch_types=[
            pltpu.VMEM((nslots,), jnp.int32),
            pltpu.VMEM((nslots,), jnp.float32),
            pltpu.VMEM((NR,), jnp.int32),
            pltpu.VMEM((NR,), jnp.float32),
            pltpu.VMEM((64,), jnp.int32),
        ],
        compiler_params=sc_params,
    )(sel_flat, wn_flat)

    xs = pl.kernel(
        _gather_body,
        out_type=jax.ShapeDtypeStruct((NR, d), jnp.float32),
        mesh=mesh,
        scratch_types=[
            pltpu.VMEM((BLK,), jnp.int32),
            pltpu.VMEM((32,), jnp.int32),
            pltpu.VMEM((32, d), jnp.float32),
            pltpu.VMEM((L,), jnp.int32),
            pltpu.SemaphoreType.DMA,
        ],
        compiler_params=sc_params,
    )(xf, slots, meta)

    w1b = w1.astype(jnp.bfloat16)
    w2b = w2.astype(jnp.bfloat16)
    xs3 = xs.reshape(NB, BLK, d)
    wv3 = wvec.reshape(NB, BLK, 1)

    ys = pl.pallas_call(
        _mm_body,
        grid_spec=pltpu.PrefetchScalarGridSpec(
            num_scalar_prefetch=1,
            grid=(NB,),
            in_specs=[
                pl.BlockSpec((1, BLK, d), lambda i, m: (i, 0, 0)),
                pl.BlockSpec((d, D_FFN), lambda i, m: (0, m[i])),
                pl.BlockSpec((D_FFN, d), lambda i, m: (m[i], 0)),
                pl.BlockSpec((1, BLK, 1), lambda i, m: (i, 0, 0)),
            ],
            out_specs=pl.BlockSpec((1, BLK, d), lambda i, m: (i, 0, 0)),
        ),
        out_shape=jax.ShapeDtypeStruct((NB, BLK, d), jnp.float32),
        compiler_params=pltpu.CompilerParams(
            dimension_semantics=("arbitrary",),
        ),
    )(meta, xs3, w1b, w2b, wv3)

    buf = pl.kernel(
        _scatter_body,
        out_type=jax.ShapeDtypeStruct((nslots + 256, d), jnp.float32),
        mesh=mesh,
        scratch_types=[
            pltpu.VMEM((BLK,), jnp.int32),
            pltpu.VMEM((4, 32), jnp.int32),
            pltpu.VMEM((32, d), jnp.float32),
            pltpu.VMEM((L,), jnp.int32),
            pltpu.SemaphoreType.DMA,
        ],
        compiler_params=sc_params,
    )(ys.reshape(NR, d), slots, meta)

    buf3 = buf.reshape((nslots + 256) // 2, 2, d)
    out = pl.pallas_call(
        _pair_body,
        grid=(t // BLK,),
        in_specs=[pl.BlockSpec((BLK, 2, d), lambda i: (i, 0, 0))],
        out_specs=pl.BlockSpec((BLK, d), lambda i: (i, 0)),
        out_shape=jax.ShapeDtypeStruct((t, d), jnp.float32),
    )(buf3)

    return out.reshape(b, s, d)


# speedup vs baseline: 1.2605x; 1.1928x over previous
import functools

import jax
import jax.numpy as jnp
from jax import lax
from jax.experimental import pallas as pl
from jax.experimental.pallas import tpu as pltpu
from jax.experimental.pallas import tpu_sc as plsc

NUM_EXPERTS = 8
N_EMBD = 1024
D_FFN = 2048
BLK = 256
NB = 24
NR = NB * BLK
HB = 128
L = 16


def _router_body(x_ref, wr_ref, sel_ref, wn_ref):
    x = x_ref[...]
    wr = wr_ref[...]
    logits = lax.dot_general(
        x, wr, (((1,), (1,)), ((), ())), preferred_element_type=jnp.float32
    )
    m = jnp.max(logits, axis=-1, keepdims=True)
    ex = jnp.exp(logits - m)
    probs = ex / jnp.sum(ex, axis=-1, keepdims=True)
    e_iota = lax.broadcasted_iota(jnp.int32, probs.shape, 1)
    m1 = jnp.max(probs, axis=-1, keepdims=True)
    i1 = jnp.min(jnp.where(probs == m1, e_iota, NUM_EXPERTS), axis=-1, keepdims=True)
    masked = jnp.where(e_iota == i1, -jnp.inf, probs)
    m2 = jnp.max(masked, axis=-1, keepdims=True)
    i2 = jnp.min(jnp.where(masked == m2, e_iota, NUM_EXPERTS), axis=-1, keepdims=True)
    s = m1 + m2
    sel_ref[...] = jnp.concatenate([i1, i2], axis=1)
    wn_ref[...] = jnp.concatenate([m1 / s, m2 / s], axis=1)


def _index_body(sel_hbm, wn_hbm, slots_hbm, wvec_hbm, meta_hbm,
                selv, wnv, slotsv, wvecv, metav):
    nslots = sel_hbm.shape[0]
    nv = nslots // L
    wid = lax.axis_index("s") * 2 + lax.axis_index("c")
    iota = lax.iota(jnp.int32, L)

    @pl.when(wid < NUM_EXPERTS + 1)
    def _():
        pltpu.sync_copy(sel_hbm, selv)

        def count_step(j, acc):
            v = selv[pl.ds(j * L, L)]
            return tuple(
                acc[f] + jnp.where(v == f, 1, 0) for f in range(NUM_EXPERTS)
            )

        zero = jnp.zeros((L,), jnp.int32)
        acc = lax.fori_loop(0, nv, count_step, (zero,) * NUM_EXPERTS)
        counts = [jnp.sum(a) for a in acc]
        bcs = [(c + BLK - 1) // BLK for c in counts]

        @pl.when(wid < NUM_EXPERTS)
        def _():
            pltpu.sync_copy(wn_hbm, wnv)
            e = wid
            base = jnp.int32(0)
            for f in range(NUM_EXPERTS):
                base = base + jnp.where(f < e, bcs[f], 0)
            base = base * BLK
            my_bc = jnp.int32(0)
            for f in range(NUM_EXPERTS):
                my_bc = my_bc + jnp.where(f == e, bcs[f], 0)

            def fill_step(j, carry):
                off = base + j * L
                slotsv[pl.ds(off, L)] = nslots + ((off + iota) & 255)
                wvecv[pl.ds(off, L)] = jnp.zeros((L,), jnp.float32)
                return carry

            lax.fori_loop(0, my_bc * (BLK // L), fill_step, 0)

            def rank_step(j, cnt):
                sl = j * L + iota
                v = selv[pl.ds(j * L, L)]
                mk = v == e
                c = plsc.cumsum(jnp.where(mk, 1, 0))
                p = base + cnt + c - 1
                plsc.store_scatter(slotsv, [p], sl, mask=mk)
                wvals = wnv[pl.ds(j * L, L)]
                plsc.store_scatter(wvecv, [p], wvals, mask=mk)
                return cnt + jnp.where(mk, 1, 0).sum()

            lax.fori_loop(0, nv, rank_step, jnp.int32(0))

            def out_step(j, carry):
                off = base + j * BLK
                pltpu.sync_copy(slotsv.at[pl.ds(off, BLK)],
                                slots_hbm.at[pl.ds(off, BLK)])
                pltpu.sync_copy(wvecv.at[pl.ds(off, BLK)],
                                wvec_hbm.at[pl.ds(off, BLK)])
                return carry

            lax.fori_loop(0, my_bc, out_step, 0)

        @pl.when(wid == NUM_EXPERTS)
        def _():
            prefix = []
            run = jnp.int32(0)
            for f in range(NUM_EXPERTS):
                run = run + bcs[f]
                prefix.append(run)
            nact = prefix[-1]
            for v in range(4):
                blk = v * L + iota
                eid = jnp.zeros((L,), jnp.int32)
                for f in range(NUM_EXPERTS - 1):
                    eid = eid + jnp.where(blk >= prefix[f], 1, 0)
                eid = jnp.where(blk < nact, eid, 0)
                if v == 3:
                    eid = jnp.where(iota == 0, nact, 0)
                metav[pl.ds(v * L, L)] = eid
            pltpu.sync_copy(metav, meta_hbm)


def _gather_body(xf_hbm, slots_hbm, meta_hbm, xs_hbm,
                 sblk, tik, rb0, rb1, metav, sem0, sem1):
    t_tokens = xf_hbm.shape[0]
    wid = lax.axis_index("s") * 2 + lax.axis_index("c")
    pltpu.sync_copy(meta_hbm.at[pl.ds(48, L)], metav)
    nhalf = 2 * jnp.max(metav[...])
    rbs = (rb0, rb1)
    sems = (sem0, sem1)

    for rep in range(2):
        h = wid + rep * 32

        @pl.when(h < nhalf)
        def _():
            pltpu.sync_copy(slots_hbm.at[pl.ds(h * HB, HB)], sblk)
            for c in range(4):
                for q in range(2):
                    sv = sblk[pl.ds(c * 32 + q * L, L)]
                    tv = jnp.minimum(lax.shift_right_logical(sv, 1),
                                     t_tokens - 1)
                    tik[c, pl.ds(q * L, L)] = tv
            cps = [None, None]
            cps[0] = pltpu.async_copy(xf_hbm.at[tik.at[0]], rb0, sem0)
            for c in range(4):
                if c < 3:
                    cps[(c + 1) % 2] = pltpu.async_copy(
                        xf_hbm.at[tik.at[c + 1]], rbs[(c + 1) % 2],
                        sems[(c + 1) % 2])
                cps[c % 2].wait()
                pltpu.sync_copy(
                    rbs[c % 2], xs_hbm.at[pl.ds(h * HB + c * 32, 32)])


def _mm_body(meta_ref, xs_ref, w1_ref, w2_ref, wv_ref, out_ref):
    b = pl.program_id(0)
    nact = meta_ref[48]

    @pl.when(b < nact)
    def _():
        h = lax.dot_general(
            xs_ref[0], w1_ref[...], (((1,), (0,)), ((), ())),
            preferred_element_type=jnp.float32,
        )
        y = lax.dot_general(
            h, w2_ref[...], (((1,), (0,)), ((), ())),
            preferred_element_type=jnp.float32,
        )
        out_ref[0] = y * wv_ref[0]


def _scatter_body(ys_hbm, slots_hbm, meta_hbm, buf_hbm,
                  sblk, sidx, rb0, rb1, metav, sem0, sem1, osem0, osem1):
    nslots = buf_hbm.shape[0] - 256
    t_tokens = nslots // 2
    wid = lax.axis_index("s") * 2 + lax.axis_index("c")
    pltpu.sync_copy(meta_hbm.at[pl.ds(48, L)], metav)
    nhalf = 2 * jnp.max(metav[...])
    rbs = (rb0, rb1)
    sems = (sem0, sem1)
    osems = (osem0, osem1)

    for rep in range(2):
        h = wid + rep * 32

        @pl.when(h < nhalf)
        def _():
            pltpu.sync_copy(slots_hbm.at[pl.ds(h * HB, HB)], sblk)
            for c in range(4):
                for q in range(2):
                    s = sblk[pl.ds(c * 32 + q * L, L)]
                    dst = jnp.where(
                        s < nslots, (s & 1) * t_tokens
                        + lax.shift_right_logical(s, 1), s)
                    sidx[c, pl.ds(q * L, L)] = dst
            cps = [None, None]
            ops = [None, None]
            cps[0] = pltpu.async_copy(
                ys_hbm.at[pl.ds(h * HB, 32)], rb0, sem0)
            for c in range(4):
                if c < 3:
                    if ops[(c + 1) % 2] is not None:
                        ops[(c + 1) % 2].wait()
                        ops[(c + 1) % 2] = None
                    cps[(c + 1) % 2] = pltpu.async_copy(
                        ys_hbm.at[pl.ds(h * HB + (c + 1) * 32, 32)],
                        rbs[(c + 1) % 2], sems[(c + 1) % 2])
                cps[c % 2].wait()
                ops[c % 2] = pltpu.async_copy(
                    rbs[c % 2], buf_hbm.at[sidx.at[c]], osems[c % 2])
            for q in range(2):
                if ops[q] is not None:
                    ops[q].wait()


def _pair_body(a_ref, b_ref, out_ref):
    out_ref[...] = a_ref[...] + b_ref[...]


def kernel(x, w_router, w1, w2):
    b, s, d = x.shape
    t = b * s
    nslots = 2 * t
    xf = x.reshape(t, d)

    sel, wn = pl.pallas_call(
        _router_body,
        out_shape=(
            jax.ShapeDtypeStruct((t, 2), jnp.int32),
            jax.ShapeDtypeStruct((t, 2), jnp.float32),
        ),
    )(xf, w_router)

    sel_flat = sel.reshape(nslots)
    wn_flat = wn.reshape(nslots)

    mesh = plsc.VectorSubcoreMesh(core_axis_name="c", subcore_axis_name="s")
    sc_params = pltpu.CompilerParams(needs_layout_passes=False)

    slots, wvec, meta = pl.kernel(
        _index_body,
        out_type=(
            jax.ShapeDtypeStruct((NR,), jnp.int32),
            jax.ShapeDtypeStruct((NR,), jnp.float32),
            jax.ShapeDtypeStruct((64,), jnp.int32),
        ),
        mesh=mesh,
        scratch_types=[
            pltpu.VMEM((nslots,), jnp.int32),
            pltpu.VMEM((nslots,), jnp.float32),
            pltpu.VMEM((NR,), jnp.int32),
            pltpu.VMEM((NR,), jnp.float32),
            pltpu.VMEM((64,), jnp.int32),
        ],
        compiler_params=sc_params,
    )(sel_flat, wn_flat)

    xs = pl.kernel(
        _gather_body,
        out_type=jax.ShapeDtypeStruct((NR, d), jnp.float32),
        mesh=mesh,
        scratch_types=[
            pltpu.VMEM((HB,), jnp.int32),
            pltpu.VMEM((4, 32), jnp.int32),
            pltpu.VMEM((32, d), jnp.float32),
            pltpu.VMEM((32, d), jnp.float32),
            pltpu.VMEM((L,), jnp.int32),
            pltpu.SemaphoreType.DMA,
            pltpu.SemaphoreType.DMA,
        ],
        compiler_params=sc_params,
    )(xf, slots, meta)

    xs3 = xs.reshape(NB, BLK, d)
    wv3 = wvec.reshape(NB, BLK, 1)

    ys = pl.pallas_call(
        _mm_body,
        grid_spec=pltpu.PrefetchScalarGridSpec(
            num_scalar_prefetch=1,
            grid=(NB,),
            in_specs=[
                pl.BlockSpec((1, BLK, d), lambda i, m: (i, 0, 0)),
                pl.BlockSpec((d, D_FFN), lambda i, m: (0, m[i])),
                pl.BlockSpec((D_FFN, d), lambda i, m: (m[i], 0)),
                pl.BlockSpec((1, BLK, 1), lambda i, m: (i, 0, 0)),
            ],
            out_specs=pl.BlockSpec((1, BLK, d), lambda i, m: (i, 0, 0)),
        ),
        out_shape=jax.ShapeDtypeStruct((NB, BLK, d), jnp.float32),
        compiler_params=pltpu.CompilerParams(
            dimension_semantics=("arbitrary",),
        ),
    )(meta, xs3, w1, w2, wv3)

    buf = pl.kernel(
        _scatter_body,
        out_type=jax.ShapeDtypeStruct((nslots + 256, d), jnp.float32),
        mesh=mesh,
        scratch_types=[
            pltpu.VMEM((HB,), jnp.int32),
            pltpu.VMEM((4, 32), jnp.int32),
            pltpu.VMEM((32, d), jnp.float32),
            pltpu.VMEM((32, d), jnp.float32),
            pltpu.VMEM((L,), jnp.int32),
            pltpu.SemaphoreType.DMA,
            pltpu.SemaphoreType.DMA,
            pltpu.SemaphoreType.DMA,
            pltpu.SemaphoreType.DMA,
        ],
        compiler_params=sc_params,
    )(ys.reshape(NR, d), slots, meta)

    nt = t // HB
    out = pl.pallas_call(
        _pair_body,
        grid=(nt,),
        in_specs=[
            pl.BlockSpec((HB, d), lambda i: (i, 0)),
            pl.BlockSpec((HB, d), lambda i: (nt + i, 0)),
        ],
        out_specs=pl.BlockSpec((HB, d), lambda i: (i, 0)),
        out_shape=jax.ShapeDtypeStruct((t, d), jnp.float32),
    )(buf, buf)

    return out.reshape(b, s, d)
